# baseline (device time: 59878 ns/iter reference)
import jax
import jax.numpy as jnp
from jax import lax
from jax.experimental import pallas as pl
from jax.experimental.pallas import tpu as pltpu

N_DEV = 8


def kernel(x, router_W, route_idx, expert_W, shared_W):
    n, d = x.shape
    n_exp_local, _, h = expert_W.shape
    n_exp = router_W.shape[1]
    chunk = n // N_DEV

    def body(x_ref, rw_ref, idx_ref, ew_ref, sw_ref, out_ref,
             partial_ref, recv_ref, send_sems, recv_sems):
        me = lax.axis_index("i")

        xv = x_ref[...]
        scores = jnp.dot(xv, rw_ref[...], preferred_element_type=jnp.float32)
        s_max = jnp.max(scores, axis=1, keepdims=True)
        ez = jnp.exp(scores - s_max)
        probs = ez / jnp.sum(ez, axis=1, keepdims=True)
        idx = idx_ref[...]
        eids = lax.broadcasted_iota(jnp.int32, (n, n_exp), 1)
        p = jnp.sum(jnp.where(eids == idx, probs, 0.0), axis=1)

        base = me * n_exp_local
        jcol = lax.broadcasted_iota(jnp.int32, (n, n_exp_local), 1) + base
        cmask = (idx == jcol).astype(jnp.float32)
        coeff = cmask * p[:, None]

        xcat = (xv[:, None, :] * coeff[:, :, None]).reshape(n, n_exp_local * d)
        ew_flat = ew_ref[...].reshape(n_exp_local * d, h)
        partial_ref[...] = jnp.dot(xcat, ew_flat,
                                   preferred_element_type=jnp.float32)

        descs = []
        for k in range(1, N_DEV):
            dst = lax.rem(me + k, N_DEV)
            slot = N_DEV - 1 - k
            rdma = pltpu.make_async_remote_copy(
                src_ref=partial_ref.at[pl.ds(dst * chunk, chunk), :],
                dst_ref=recv_ref.at[slot],
                send_sem=send_sems.at[k - 1],
                recv_sem=recv_sems.at[slot],
                device_id=(dst,),
                device_id_type=pl.DeviceIdType.MESH,
            )
            rdma.start()
            descs.append((rdma, slot))

        xs = x_ref[pl.ds(me * chunk, chunk), :]
        out_ref[...] = (
            jnp.dot(xs, sw_ref[...], preferred_element_type=jnp.float32)
            + partial_ref[pl.ds(me * chunk, chunk), :]
        )

        for rdma, slot in descs:
            rdma.wait_recv()
            out_ref[...] += recv_ref[slot]
        for rdma, _ in descs:
            rdma.wait_send()

    return pl.pallas_call(
        body,
        out_shape=jax.ShapeDtypeStruct((chunk, h), jnp.float32),
        in_specs=[pl.BlockSpec(memory_space=pltpu.VMEM)] * 5,
        out_specs=pl.BlockSpec(memory_space=pltpu.VMEM),
        scratch_shapes=[
            pltpu.VMEM((n, h), jnp.float32),
            pltpu.VMEM((N_DEV - 1, chunk, h), jnp.float32),
            pltpu.SemaphoreType.DMA((N_DEV - 1,)),
            pltpu.SemaphoreType.DMA((N_DEV - 1,)),
        ],
    )(x, router_W, route_idx, expert_W, shared_W)


# device time: 42265 ns/iter; 1.4167x vs baseline; 1.4167x over previous
import jax
import jax.numpy as jnp
from jax import lax
from jax.experimental import pallas as pl
from jax.experimental.pallas import tpu as pltpu

N_DEV = 8


def kernel(x, router_W, route_idx, expert_W, shared_W):
    n, d = x.shape
    n_exp_local, _, h = expert_W.shape
    n_exp = router_W.shape[1]
    chunk = n // N_DEV

    def body(x_ref, rw_ref, idx_ref, ew_ref, sw_ref, out_ref,
             partial_ref, recv_ref, send_sems, recv_sems):
        me = lax.axis_index("i")

        xv = x_ref[...]
        scores = jnp.dot(xv, rw_ref[...], preferred_element_type=jnp.float32)
        s_max = jnp.max(scores, axis=1, keepdims=True)
        ez = jnp.exp(scores - s_max)
        probs = ez / jnp.sum(ez, axis=1, keepdims=True)
        idx = idx_ref[...]
        eids = lax.broadcasted_iota(jnp.int32, (n, n_exp), 1)
        p = jnp.sum(jnp.where(eids == idx, probs, 0.0), axis=1)

        base = me * n_exp_local
        jcol = lax.broadcasted_iota(jnp.int32, (n, n_exp_local), 1) + base
        cmask = (idx == jcol).astype(jnp.float32)
        coeff = cmask * p[:, None]

        xcat = ((xv[:, None, :] * coeff[:, :, None])
                .reshape(n, n_exp_local * d).astype(jnp.bfloat16))
        ew_flat = ew_ref[...].reshape(n_exp_local * d, h).astype(jnp.bfloat16)
        partial_ref[...] = jnp.dot(xcat, ew_flat,
                                   preferred_element_type=jnp.float32
                                   ).astype(jnp.bfloat16)

        descs = []
        for k in range(1, N_DEV):
            dst = lax.rem(me + k, N_DEV)
            slot = N_DEV - 1 - k
            rdma = pltpu.make_async_remote_copy(
                src_ref=partial_ref.at[pl.ds(dst * chunk, chunk), :],
                dst_ref=recv_ref.at[slot],
                send_sem=send_sems.at[k - 1],
                recv_sem=recv_sems.at[slot],
                device_id=(dst,),
                device_id_type=pl.DeviceIdType.MESH,
            )
            rdma.start()
            descs.append((rdma, slot))

        xs = x_ref[pl.ds(me * chunk, chunk), :]
        out_ref[...] = (
            jnp.dot(xs, sw_ref[...], preferred_element_type=jnp.float32)
            + partial_ref[pl.ds(me * chunk, chunk), :].astype(jnp.float32)
        )

        for rdma, slot in descs:
            rdma.wait_recv()
            out_ref[...] += recv_ref[slot].astype(jnp.float32)
        for rdma, _ in descs:
            rdma.wait_send()

    return pl.pallas_call(
        body,
        out_shape=jax.ShapeDtypeStruct((chunk, h), jnp.float32),
        in_specs=[pl.BlockSpec(memory_space=pltpu.VMEM)] * 5,
        out_specs=pl.BlockSpec(memory_space=pltpu.VMEM),
        scratch_shapes=[
            pltpu.VMEM((n, h), jnp.bfloat16),
            pltpu.VMEM((N_DEV - 1, chunk, h), jnp.bfloat16),
            pltpu.SemaphoreType.DMA((N_DEV - 1,)),
            pltpu.SemaphoreType.DMA((N_DEV - 1,)),
        ],
    )(x, router_W, route_idx, expert_W, shared_W)


# device time: 36016 ns/iter; 1.6625x vs baseline; 1.1735x over previous
import jax
import jax.numpy as jnp
from jax import lax
from jax.experimental import pallas as pl
from jax.experimental.pallas import tpu as pltpu

N_DEV = 8


def kernel(x, router_W, route_idx, expert_W, shared_W):
    n, d = x.shape
    n_exp_local, _, h = expert_W.shape
    n_exp = router_W.shape[1]
    chunk = n // N_DEV

    def body(x_ref, rw_ref, idx_ref, ew_ref, sw_ref, out_ref,
             xb_ref, cb_ref, send_ref, recv_ref, send_sems, recv_sems):
        me = lax.axis_index("i")

        xv = x_ref[...]
        scores = jnp.dot(xv, rw_ref[...], preferred_element_type=jnp.float32)
        s_max = jnp.max(scores, axis=1, keepdims=True)
        ez = jnp.exp(scores - s_max)
        probs = ez / jnp.sum(ez, axis=1, keepdims=True)
        idx = idx_ref[...]
        eids = lax.broadcasted_iota(jnp.int32, (n, n_exp), 1)
        p = jnp.sum(jnp.where(eids == idx, probs, 0.0), axis=1)

        base = me * n_exp_local
        jcol = lax.broadcasted_iota(jnp.int32, (n, n_exp_local), 1) + base
        cmask = (idx == jcol).astype(jnp.float32)
        coeff = cmask * p[:, None]

        xb_ref[...] = xv.astype(jnp.bfloat16)
        cb_ref[...] = coeff.astype(jnp.bfloat16)
        ew_flat = ew_ref[...].reshape(n_exp_local * d, h).astype(jnp.bfloat16)

        def chunk_partial(pos):
            xc = xb_ref[pl.ds(pos * chunk, chunk), :]
            cc = cb_ref[pl.ds(pos * chunk, chunk), :]
            xcat = (xc[:, None, :] * cc[:, :, None]).reshape(
                chunk, n_exp_local * d)
            return jnp.dot(xcat, ew_flat, preferred_element_type=jnp.float32)

        descs = []
        for k in range(1, N_DEV):
            dst = lax.rem(me + k, N_DEV)
            slot = N_DEV - 1 - k
            send_ref[k - 1] = chunk_partial(dst).astype(jnp.bfloat16)
            rdma = pltpu.make_async_remote_copy(
                src_ref=send_ref.at[k - 1],
                dst_ref=recv_ref.at[slot],
                send_sem=send_sems.at[k - 1],
                recv_sem=recv_sems.at[slot],
                device_id=(dst,),
                device_id_type=pl.DeviceIdType.MESH,
            )
            rdma.start()
            descs.append((rdma, slot))

        xs = x_ref[pl.ds(me * chunk, chunk), :]
        out_ref[...] = (
            jnp.dot(xs, sw_ref[...], preferred_element_type=jnp.float32)
            + chunk_partial(me)
        )

        for rdma, slot in descs:
            rdma.wait_recv()
            out_ref[...] += recv_ref[slot].astype(jnp.float32)
        for rdma, _ in descs:
            rdma.wait_send()

    return pl.pallas_call(
        body,
        out_shape=jax.ShapeDtypeStruct((chunk, h), jnp.float32),
        in_specs=[pl.BlockSpec(memory_space=pltpu.VMEM)] * 5,
        out_specs=pl.BlockSpec(memory_space=pltpu.VMEM),
        scratch_shapes=[
            pltpu.VMEM((n, d), jnp.bfloat16),
            pltpu.VMEM((n, n_exp_local), jnp.bfloat16),
            pltpu.VMEM((N_DEV - 1, chunk, h), jnp.bfloat16),
            pltpu.VMEM((N_DEV - 1, chunk, h), jnp.bfloat16),
            pltpu.SemaphoreType.DMA((N_DEV - 1,)),
            pltpu.SemaphoreType.DMA((N_DEV - 1,)),
        ],
    )(x, router_W, route_idx, expert_W, shared_W)


# device time: 26325 ns/iter; 2.2746x vs baseline; 1.3681x over previous
import jax
import jax.numpy as jnp
from jax import lax
from jax.experimental import pallas as pl
from jax.experimental.pallas import tpu as pltpu

N_DEV = 8
PLANE = 4
PAD_A = 112
PAD_B = 64


def kernel(x, router_W, route_idx, expert_W, shared_W):
    n, d = x.shape
    n_exp_local, _, h = expert_W.shape
    n_exp = router_W.shape[1]
    chunk = n // N_DEV
    half = n // 2

    def body(x_ref, rw_ref, idx_ref, ew_ref, sw_ref, out_ref,
             xb_ref, cb_ref, acc_ref, sendA_ref, recvA_ref,
             sendB_ref, recvB_ref, send_sems, recv_sems):
        me = lax.axis_index("i")
        f32 = jnp.float32
        bf16 = jnp.bfloat16

        partner = lax.rem(me + PLANE, N_DEV)
        plane = me // PLANE
        mine_start = plane * half
        other_start = (1 - plane) * half
        mp = lax.rem(me, PLANE)

        bsem = pltpu.get_barrier_semaphore()
        peers = [partner] + [
            plane * PLANE + lax.rem(mp + t, PLANE) for t in (1, 2, 3)]
        for pr in peers:
            pl.semaphore_signal(bsem, inc=1, device_id=(pr,),
                                device_id_type=pl.DeviceIdType.MESH)

        xv = x_ref[...]
        scores = jnp.dot(xv, rw_ref[...], preferred_element_type=f32)
        s_max = jnp.max(scores, axis=1, keepdims=True)
        ez = jnp.exp(scores - s_max)
        probs = ez / jnp.sum(ez, axis=1, keepdims=True)
        idx = idx_ref[...]
        eids = lax.broadcasted_iota(jnp.int32, (n, n_exp), 1)
        p = jnp.sum(jnp.where(eids == idx, probs, 0.0), axis=1)

        base = me * n_exp_local
        jcol = lax.broadcasted_iota(jnp.int32, (n, n_exp_local), 1) + base
        cmask = (idx == jcol).astype(f32)
        coeff = cmask * p[:, None]

        xb_ref[...] = xv.astype(bf16)
        cb_ref[...] = coeff.astype(bf16)
        ew_flat = ew_ref[...].reshape(n_exp_local * d, h).astype(bf16)

        def contract0(a, b):
            return lax.dot_general(a, b, (((0,), (0,)), ((), ())),
                                   preferred_element_type=f32)

        def onehot_rank(maskf, length, pad):
            tri = (lax.broadcasted_iota(jnp.int32, (length, length), 0)
                   >= lax.broadcasted_iota(jnp.int32, (length, length), 1)
                   ).astype(f32)
            rank = jnp.dot(tri, maskf, preferred_element_type=f32) - 1.0
            lane = lax.broadcasted_iota(jnp.int32, (length, pad), 1
                                        ).astype(f32)
            return (jnp.where(maskf > 0, rank, -1.0) == lane).astype(bf16)

        def route_dev(start, length):
            return idx_ref[pl.ds(start, length), :] // n_exp_local

        def sparse_partial(start, length, ot, pad):
            xc = xb_ref[pl.ds(start, length), :]
            cc = cb_ref[pl.ds(start, length), :]
            xq = contract0(ot, xc).astype(bf16)
            cq = contract0(ot, cc).astype(bf16)
            xcat = (xq[:, None, :] * cq[:, :, None]).reshape(
                pad, n_exp_local * d)
            return jnp.dot(xcat, ew_flat, preferred_element_type=f32)

        pl.semaphore_wait(bsem, len(peers))

        rd_other = route_dev(other_start, half)
        otA = onehot_rank((rd_other == me).astype(f32), half, PAD_A)
        sendA_ref[...] = sparse_partial(other_start, half, otA,
                                        PAD_A).astype(bf16)
        rdmaA = pltpu.make_async_remote_copy(
            src_ref=sendA_ref, dst_ref=recvA_ref,
            send_sem=send_sems.at[3], recv_sem=recv_sems.at[3],
            device_id=(partner,), device_id_type=pl.DeviceIdType.MESH,
        )
        rdmaA.start()

        rd_mine = route_dev(mine_start, half)
        otM = onehot_rank((rd_mine == me).astype(f32), half, PAD_A)
        partM = sparse_partial(mine_start, half, otM, PAD_A)
        acc_ref[...] = jnp.dot(otM, partM.astype(bf16),
                               preferred_element_type=f32).astype(bf16)

        rdmaA.wait_recv()
        otAr = onehot_rank((rd_mine == partner).astype(f32), half, PAD_A)
        acc_ref[...] += jnp.dot(otAr, recvA_ref[...],
                                preferred_element_type=f32).astype(bf16)

        descsB = []
        for t in (1, 2, 3):
            dst_local = lax.rem(mp + t, PLANE)
            rd_w = route_dev(mine_start + dst_local * chunk, chunk)
            otB = onehot_rank((lax.rem(rd_w, PLANE) == mp).astype(f32),
                              chunk, PAD_B)
            accw = acc_ref[pl.ds(dst_local * chunk, chunk), :]
            sendB_ref[t - 1] = contract0(otB, accw).astype(bf16)
            rdma = pltpu.make_async_remote_copy(
                src_ref=sendB_ref.at[t - 1],
                dst_ref=recvB_ref.at[3 - t],
                send_sem=send_sems.at[t - 1],
                recv_sem=recv_sems.at[3 - t],
                device_id=(plane * PLANE + dst_local,),
                device_id_type=pl.DeviceIdType.MESH,
            )
            rdma.start()
            descsB.append((rdma, 3 - t))

        xs = x_ref[pl.ds(me * chunk, chunk), :]
        out_ref[...] = (
            jnp.dot(xs, sw_ref[...], preferred_element_type=f32)
            + acc_ref[pl.ds(mp * chunk, chunk), :].astype(f32)
        )

        rd_me = route_dev(me * chunk, chunk)
        ots_recv = []
        for t in (1, 2, 3):
            src_local = lax.rem(mp - t + PLANE, PLANE)
            ots_recv.append(onehot_rank(
                (lax.rem(rd_me, PLANE) == src_local).astype(f32),
                chunk, PAD_B))
        for (rdma, slot), otBr in zip(descsB, ots_recv):
            rdma.wait_recv()
            out_ref[...] += jnp.dot(otBr, recvB_ref[slot],
                                    preferred_element_type=f32)

        rdmaA.wait_send()
        for rdma, _ in descsB:
            rdma.wait_send()

    return pl.pallas_call(
        body,
        out_shape=jax.ShapeDtypeStruct((chunk, h), jnp.float32),
        in_specs=[pl.BlockSpec(memory_space=pltpu.VMEM)] * 5,
        out_specs=pl.BlockSpec(memory_space=pltpu.VMEM),
        scratch_shapes=[
            pltpu.VMEM((n, d), jnp.bfloat16),
            pltpu.VMEM((n, n_exp_local), jnp.bfloat16),
            pltpu.VMEM((half, h), jnp.bfloat16),
            pltpu.VMEM((PAD_A, h), jnp.bfloat16),
            pltpu.VMEM((PAD_A, h), jnp.bfloat16),
            pltpu.VMEM((3, PAD_B, h), jnp.bfloat16),
            pltpu.VMEM((3, PAD_B, h), jnp.bfloat16),
            pltpu.SemaphoreType.DMA((4,)),
            pltpu.SemaphoreType.DMA((4,)),
        ],
        compiler_params=pltpu.CompilerParams(collective_id=0),
    )(x, router_W, route_idx, expert_W, shared_W)
